# baseline (speedup 1.0000x reference)
import functools

import jax
import jax.numpy as jnp
from jax import lax
from jax.experimental import pallas as pl
from jax.experimental.pallas import tpu as pltpu
from jax.experimental.pallas import tpu_sc as plsc

NUM_CORES = 2
NUM_SUBCORES = 16
NUM_WORKERS = NUM_CORES * NUM_SUBCORES

LANES = 16
SUBLANES = 8
PAIR_ILV = 2
D_ILV = 4


def _sc_embed_tiled(idx_w, table_flat, dim):
    _, hist, bcol = idx_w.shape
    stride = dim + 1
    n_bvecs = bcol // LANES
    d_hi = dim // SUBLANES
    assert hist % 4 == 0 and dim % (SUBLANES * D_ILV) == 0

    mesh = plsc.VectorSubcoreMesh(core_axis_name="c", subcore_axis_name="s")

    @functools.partial(
        pl.kernel,
        out_type=jax.ShapeDtypeStruct(
            (hist * d_hi, NUM_WORKERS, SUBLANES, 128), jnp.float32),
        mesh=mesh,
        scratch_types=[
            pltpu.VMEM(table_flat.shape, jnp.float32),
            pltpu.VMEM((hist, bcol), jnp.int32),
            pltpu.VMEM((d_hi, SUBLANES, 128), jnp.float32),
            pltpu.VMEM((d_hi, SUBLANES, 128), jnp.float32),
            pltpu.VMEM((d_hi, SUBLANES, 128), jnp.float32),
            pltpu.VMEM((d_hi, SUBLANES, 128), jnp.float32),
            pltpu.SemaphoreType.DMA,
            pltpu.SemaphoreType.DMA,
            pltpu.SemaphoreType.DMA,
            pltpu.SemaphoreType.DMA,
        ],
        compiler_params=pltpu.CompilerParams(use_tc_tiling_on_sc=False,
                                             needs_layout_passes=False),
    )
    def k(table_hbm, idxt_hbm, out_hbm, table_v, idx_v, buf0, buf1,
          buf2, buf3, o0sem, o1sem, o2sem, o3sem):
        wid = lax.axis_index("s") * NUM_CORES + lax.axis_index("c")

        pltpu.sync_copy(table_hbm, table_v)
        pltpu.sync_copy(idxt_hbm.at[wid], idx_v)

        def compute(t, buf):
            @plsc.parallel_loop(0, n_bvecs, step=PAIR_ILV)
            def v_body(v0):
                bases = []
                for p in range(PAIR_ILV):
                    idx16 = idx_v[t, pl.ds((v0 + p) * LANES, LANES)]
                    bases.append(idx16 * stride)
                for d0 in range(0, dim, D_ILV):
                    vals = [
                        plsc.load_gather(table_v, [bases[p] + (d0 + j)])
                        for j in range(D_ILV) for p in range(PAIR_ILV)
                    ]
                    i = 0
                    for j in range(D_ILV):
                        d = d0 + j
                        for p in range(PAIR_ILV):
                            buf[d // SUBLANES, d % SUBLANES,
                                pl.ds((v0 + p) * LANES, LANES)] = vals[i]
                            i += 1

        def fire_out(t, buf, sem):
            pltpu.async_copy(
                buf, out_hbm.at[pl.ds(t * d_hi, d_hi), wid], sem)

        def wait_out(buf, sem):
            pltpu.make_async_copy(
                buf, out_hbm.at[pl.ds(0, d_hi), 0], sem).wait()

        bufs = (buf0, buf1, buf2, buf3)
        sems = (o0sem, o1sem, o2sem, o3sem)

        def body(i, carry):
            @pl.when(i > 0)
            def _():
                for buf, sem in zip(bufs, sems):
                    wait_out(buf, sem)

            for q, (buf, sem) in enumerate(zip(bufs, sems)):
                compute(4 * i + q, buf)
                fire_out(4 * i + q, buf, sem)
            return carry

        lax.fori_loop(0, hist // 4, body, 0)
        for buf, sem in zip(bufs, sems):
            wait_out(buf, sem)

    return k(table_flat, idx_w)


def kernel(inputs, table):
    b, t = inputs.shape
    vocab, dim = table.shape
    idx_w = inputs.astype(jnp.int32).T.reshape(
        t, NUM_WORKERS, b // NUM_WORKERS).transpose(1, 0, 2)
    table_flat = jnp.pad(table, ((0, 0), (0, 1))).reshape(-1)
    out4 = _sc_embed_tiled(idx_w, table_flat, dim)
    out5 = out4.reshape(t, dim // 8, b // 128, 8, 128)
    return out5.transpose(2, 4, 0, 1, 3).reshape(b, t, dim)


# speedup vs baseline: 1.2548x; 1.2548x over previous
import functools

import jax
import jax.numpy as jnp
from jax import lax
from jax.experimental import pallas as pl
from jax.experimental.pallas import tpu as pltpu
from jax.experimental.pallas import tpu_sc as plsc

NUM_CORES = 2
NUM_SUBCORES = 16
NUM_WORKERS = NUM_CORES * NUM_SUBCORES

LANES = 16
SUBLANES = 8
PAIR_ILV = 1
D_ILV = 8


def _sc_embed_tiled(idx_w, table_flat, dim):
    _, hist, bcol = idx_w.shape
    stride = dim + 1
    n_bvecs = bcol // LANES
    d_hi = dim // SUBLANES
    assert hist % 2 == 0 and dim % (SUBLANES * D_ILV) == 0

    mesh = plsc.VectorSubcoreMesh(core_axis_name="c", subcore_axis_name="s")

    @functools.partial(
        pl.kernel,
        out_type=jax.ShapeDtypeStruct(
            (hist * d_hi, NUM_WORKERS, SUBLANES, 128), jnp.float32),
        mesh=mesh,
        scratch_types=[
            pltpu.VMEM(table_flat.shape, jnp.float32),
            pltpu.VMEM((hist, bcol), jnp.int32),
            pltpu.VMEM((d_hi, SUBLANES, 128), jnp.float32),
            pltpu.VMEM((d_hi, SUBLANES, 128), jnp.float32),
            pltpu.SemaphoreType.DMA,
            pltpu.SemaphoreType.DMA,
        ],
        compiler_params=pltpu.CompilerParams(use_tc_tiling_on_sc=False,
                                             needs_layout_passes=False),
    )
    def k(table_hbm, idxt_hbm, out_hbm, table_v, idx_v, buf0, buf1,
          o0sem, o1sem):
        wid = lax.axis_index("s") * NUM_CORES + lax.axis_index("c")

        pltpu.sync_copy(table_hbm, table_v)
        pltpu.sync_copy(idxt_hbm.at[wid], idx_v)

        def compute(t, buf):
            @plsc.parallel_loop(0, n_bvecs, step=PAIR_ILV)
            def v_body(v0):
                bases = []
                for p in range(PAIR_ILV):
                    idx16 = idx_v[t, pl.ds((v0 + p) * LANES, LANES)]
                    bases.append(idx16 * stride)
                for d0 in range(0, dim, D_ILV):
                    vals = [
                        plsc.load_gather(table_v, [bases[p] + (d0 + j)])
                        for j in range(D_ILV) for p in range(PAIR_ILV)
                    ]
                    i = 0
                    for j in range(D_ILV):
                        d = d0 + j
                        for p in range(PAIR_ILV):
                            buf[d // SUBLANES, d % SUBLANES,
                                pl.ds((v0 + p) * LANES, LANES)] = vals[i]
                            i += 1

        def fire_out(t, buf, sem):
            pltpu.async_copy(
                buf, out_hbm.at[pl.ds(t * d_hi, d_hi), wid], sem)

        def wait_out(buf, sem):
            pltpu.make_async_copy(
                buf, out_hbm.at[pl.ds(0, d_hi), 0], sem).wait()

        def body(i, carry):
            @pl.when(i > 0)
            def _():
                wait_out(buf0, o0sem)
                wait_out(buf1, o1sem)

            compute(2 * i, buf0)
            fire_out(2 * i, buf0, o0sem)
            compute(2 * i + 1, buf1)
            fire_out(2 * i + 1, buf1, o1sem)
            return carry

        lax.fori_loop(0, hist // 2, body, 0)
        wait_out(buf0, o0sem)
        wait_out(buf1, o1sem)

    return k(table_flat, idx_w)


def kernel(inputs, table):
    b, t = inputs.shape
    vocab, dim = table.shape
    idx_w = inputs.astype(jnp.int32).T.reshape(
        t, NUM_WORKERS, b // NUM_WORKERS).transpose(1, 0, 2)
    table_flat = jnp.pad(table, ((0, 0), (0, 1))).reshape(-1)
    out4 = _sc_embed_tiled(idx_w, table_flat, dim)
    out5 = out4.reshape(t, dim // 8, b // 128, 8, 128)
    return out5.transpose(2, 4, 0, 1, 3).reshape(b, t, dim)
